# baseline (speedup 1.0000x reference)
import functools

import jax
import jax.numpy as jnp
from jax import lax
from jax.experimental import pallas as pl
from jax.experimental.pallas import tpu as pltpu
from jax.experimental.pallas import tpu_sc as plsc

N = 10000
E = 320000
D = 128

NSC = 2
NSUB = 16
CHUNK = 32
CHUNKS_PER_WORKER = 320
EPW = CHUNK * CHUNKS_PER_WORKER
E_PAD = NSC * NSUB * EPW
IDX_ROWS = E_PAD // CHUNK
NP = N + NSUB
ZROWS = 624
ZTAIL = NP - (NSUB - 1) * ZROWS
OROWS = 624
OTAIL = N - (NSUB - 1) * OROWS

R = 1000
GRID = N // R


NBUF = 8
IDX_PIECES = 8
IDX_HALF = CHUNKS_PER_WORKER // IDX_PIECES


def _sc_scatter_body(m_hbm, src_hbm, dst_hbm, zeros_hbm, out0, out1, *scr):
    c = lax.axis_index("c")
    s = lax.axis_index("s")
    wid = c * NSUB + s
    src_v, dst_v = scr[0], scr[1]
    bufs = scr[2:2 + NBUF]
    acc = scr[2 + NBUF]
    sgs = scr[3 + NBUF:3 + 2 * NBUF]
    sss = scr[3 + 2 * NBUF:3 + 3 * NBUF]

    @pl.when(s < NSUB - 1)
    def _():
        pltpu.sync_copy(zeros_hbm.at[pl.ds(0, ZROWS)],
                        acc.at[pl.ds(s * ZROWS, ZROWS)])

    @pl.when(s == NSUB - 1)
    def _():
        pltpu.sync_copy(zeros_hbm, acc.at[pl.ds((NSUB - 1) * ZROWS, ZTAIL)])

    def _wait(sem, buf):
        pltpu.make_async_copy(m_hbm.at[pl.ds(0, CHUNK)], buf, sem).wait()

    base = wid * CHUNKS_PER_WORKER
    for h in range(IDX_PIECES):
        pltpu.sync_copy(src_hbm.at[pl.ds(base + h * IDX_HALF, IDX_HALF)],
                        src_v)
        pltpu.sync_copy(dst_hbm.at[pl.ds(base + h * IDX_HALF, IDX_HALF)],
                        dst_v)
        if h == 0:
            plsc.subcore_barrier()

        for b in range(NBUF):
            pltpu.async_copy(m_hbm.at[src_v.at[b]], bufs[b], sgs[b])

        def round_body(r, carry):
            j0 = r * NBUF
            for b in range(NBUF):
                _wait(sgs[b], bufs[b])
                pltpu.async_copy(bufs[b], acc.at[dst_v.at[j0 + b]], sss[b],
                                 add=True)
            for b in range(NBUF):
                _wait(sss[b], bufs[b])

                @pl.when(j0 + NBUF + b < IDX_HALF)
                def _():
                    pltpu.async_copy(m_hbm.at[src_v.at[j0 + NBUF + b]],
                                     bufs[b], sgs[b])
            return carry

        lax.fori_loop(0, IDX_HALF // NBUF, round_body, 0)

    plsc.subcore_barrier()

    ob = s * OROWS
    tb = (NSUB - 1) * OROWS

    @pl.when((c == 0) & (s < NSUB - 1))
    def _():
        pltpu.sync_copy(acc.at[pl.ds(ob, OROWS)], out0.at[pl.ds(ob, OROWS)])

    @pl.when((c == 0) & (s == NSUB - 1))
    def _():
        pltpu.sync_copy(acc.at[pl.ds(tb, OTAIL)], out0.at[pl.ds(tb, OTAIL)])

    @pl.when((c == 1) & (s < NSUB - 1))
    def _():
        pltpu.sync_copy(acc.at[pl.ds(ob, OROWS)], out1.at[pl.ds(ob, OROWS)])

    @pl.when((c == 1) & (s == NSUB - 1))
    def _():
        pltpu.sync_copy(acc.at[pl.ds(tb, OTAIL)], out1.at[pl.ds(tb, OTAIL)])


@functools.lru_cache(maxsize=None)
def _get_sc_scatter():
    return pl.kernel(
        _sc_scatter_body,
        out_type=[jax.ShapeDtypeStruct((N, D), jnp.float32),
                  jax.ShapeDtypeStruct((N, D), jnp.float32)],
        mesh=plsc.VectorSubcoreMesh(core_axis_name="c", subcore_axis_name="s",
                                    num_cores=NSC, num_subcores=NSUB),
        scratch_types=[
            pltpu.VMEM((IDX_HALF, CHUNK), jnp.int32),
            pltpu.VMEM((IDX_HALF, CHUNK), jnp.int32),
        ] + [pltpu.VMEM((CHUNK, D), jnp.float32) for _ in range(NBUF)] + [
            pltpu.VMEM_SHARED((NP, D), jnp.float32),
        ] + [pltpu.SemaphoreType.DMA for _ in range(2 * NBUF)],
    )


def _dot_t(x, w):
    return lax.dot_general(x, w, (((1,), (1,)), ((), ())),
                           preferred_element_type=jnp.float32)


def _m_body(h_ref, we_ref, be_ref, o_ref):
    o_ref[...] = _dot_t(h_ref[...], we_ref[...]) + be_ref[...]


def _gru_core(a, h, wi_ref, wh_ref, bi_ref, bh_ref):
    gi = _dot_t(a, wi_ref[...]) + bi_ref[...]
    gh = _dot_t(h, wh_ref[...]) + bh_ref[...]
    r = jax.nn.sigmoid(gi[:, :D] + gh[:, :D])
    z = jax.nn.sigmoid(gi[:, D:2 * D] + gh[:, D:2 * D])
    n = jnp.tanh(gi[:, 2 * D:] + r * gh[:, 2 * D:])
    return (1.0 - z) * n + z * h


def _gru_m_body(a0_ref, a1_ref, h_ref, wi_ref, wh_ref, bi_ref, bh_ref,
                wen_ref, ben_ref, h_out, m_out):
    hn = _gru_core(a0_ref[...] + a1_ref[...], h_ref[...],
                   wi_ref, wh_ref, bi_ref, bh_ref)
    h_out[...] = hn
    m_out[...] = _dot_t(hn, wen_ref[...]) + ben_ref[...]


def _gru_fc_body(a0_ref, a1_ref, h_ref, wi_ref, wh_ref, bi_ref, bh_ref,
                 wfc_ref, bfc_ref, o_ref):
    hn = _gru_core(a0_ref[...] + a1_ref[...], h_ref[...],
                   wi_ref, wh_ref, bi_ref, bh_ref)
    o_ref[...] = _dot_t(hn, wfc_ref[...]) + bfc_ref[...]


def _row_spec(width):
    return pl.BlockSpec((R, width), lambda i: (i, 0))


def _full_spec(shape):
    nd = len(shape)
    return pl.BlockSpec(shape, lambda i: (0,) * nd)


def _tc_m(h, we, be):
    return pl.pallas_call(
        _m_body,
        grid=(GRID,),
        in_specs=[_row_spec(D), _full_spec((D, D)), _full_spec((1, D))],
        out_specs=_row_spec(D),
        out_shape=jax.ShapeDtypeStruct((N, D), jnp.float32),
    )(h, we, be)


def _tc_gru_m(a0, a1, h, wi, wh, bi, bh, wen, ben):
    return pl.pallas_call(
        _gru_m_body,
        grid=(GRID,),
        in_specs=[_row_spec(D), _row_spec(D), _row_spec(D),
                  _full_spec((3 * D, D)), _full_spec((3 * D, D)),
                  _full_spec((1, 3 * D)), _full_spec((1, 3 * D)),
                  _full_spec((D, D)), _full_spec((1, D))],
        out_specs=[_row_spec(D), _row_spec(D)],
        out_shape=[jax.ShapeDtypeStruct((N, D), jnp.float32),
                   jax.ShapeDtypeStruct((N, D), jnp.float32)],
    )(a0, a1, h, wi, wh, bi, bh, wen, ben)


def _tc_gru_fc(a0, a1, h, wi, wh, bi, bh, wfc, bfc):
    return pl.pallas_call(
        _gru_fc_body,
        grid=(GRID,),
        in_specs=[_row_spec(D), _row_spec(D), _row_spec(D),
                  _full_spec((3 * D, D)), _full_spec((3 * D, D)),
                  _full_spec((1, 3 * D)), _full_spec((1, 3 * D)),
                  _full_spec((16, D)), _full_spec((1, 16))],
        out_specs=_row_spec(16),
        out_shape=jax.ShapeDtypeStruct((N, 16), jnp.float32),
    )(a0, a1, h, wi, wh, bi, bh, wfc, bfc)


def kernel(features, edge_index, W_e0, b_e0, Wi0, Wh0, bi0, bh0,
           W_e1, b_e1, Wi1, Wh1, bi1, bh1, W_fc, b_fc):
    pad = E_PAD - E
    pad_ids = jnp.arange(pad, dtype=jnp.int32)
    src = jnp.concatenate([edge_index[0], pad_ids % N]).reshape(IDX_ROWS, CHUNK)
    dst = jnp.concatenate([edge_index[1], N + (pad_ids % NSUB)]
                          ).reshape(IDX_ROWS, CHUNK)
    zeros_init = jnp.zeros((ZTAIL, D), jnp.float32)

    params = ((W_e0, b_e0.reshape(1, D), Wi0, Wh0,
               bi0.reshape(1, 3 * D), bh0.reshape(1, 3 * D)),
              (W_e1, b_e1.reshape(1, D), Wi1, Wh1,
               bi1.reshape(1, 3 * D), bh1.reshape(1, 3 * D)))

    h = features
    m = _tc_m(h, params[0][0], params[0][1])
    for layer in range(2):
        _, _, wi, wh, bi, bh = params[layer]
        for step in range(3):
            a0, a1 = _get_sc_scatter()(m, src, dst, zeros_init)
            last = layer == 1 and step == 2
            if last:
                return _tc_gru_fc(a0, a1, h, wi, wh, bi, bh,
                                  W_fc, b_fc.reshape(1, 16))
            if step == 2:
                wen, ben = params[layer + 1][0], params[layer + 1][1]
            else:
                wen, ben = params[layer][0], params[layer][1]
            h, m = _tc_gru_m(a0, a1, h, wi, wh, bi, bh, wen, ben)


# speedup vs baseline: 1.0621x; 1.0621x over previous
import functools

import jax
import jax.numpy as jnp
from jax import lax
from jax.experimental import pallas as pl
from jax.experimental.pallas import tpu as pltpu
from jax.experimental.pallas import tpu_sc as plsc

N = 10000
E = 320000
D = 128

NSC = 2
NSUB = 16
CHUNK = 64
CHUNKS_PER_WORKER = 160
EPW = CHUNK * CHUNKS_PER_WORKER
E_PAD = NSC * NSUB * EPW
IDX_ROWS = E_PAD // CHUNK
NP = N + NSUB
ZROWS = 624
ZTAIL = NP - (NSUB - 1) * ZROWS
OROWS = 624
OTAIL = N - (NSUB - 1) * OROWS

R = 1000
GRID = N // R


NBUF = 4
IDX_PIECES = 4
IDX_HALF = CHUNKS_PER_WORKER // IDX_PIECES


def _sc_scatter_body(m_hbm, src_hbm, dst_hbm, zeros_hbm, out0, out1, *scr):
    c = lax.axis_index("c")
    s = lax.axis_index("s")
    wid = c * NSUB + s
    src_v, dst_v = scr[0], scr[1]
    bufs = scr[2:2 + NBUF]
    acc = scr[2 + NBUF]
    sgs = scr[3 + NBUF:3 + 2 * NBUF]
    sss = scr[3 + 2 * NBUF:3 + 3 * NBUF]

    def _wait(sem, buf):
        pltpu.make_async_copy(m_hbm.at[pl.ds(0, CHUNK)], buf, sem).wait()

    base = wid * CHUNKS_PER_WORKER
    for h in range(IDX_PIECES):
        pltpu.sync_copy(src_hbm.at[pl.ds(base + h * IDX_HALF, IDX_HALF)],
                        src_v)
        pltpu.sync_copy(dst_hbm.at[pl.ds(base + h * IDX_HALF, IDX_HALF)],
                        dst_v)

        for b in range(NBUF):
            pltpu.async_copy(m_hbm.at[src_v.at[b]], bufs[b], sgs[b])

        if h == 0:
            @pl.when(s < NSUB - 1)
            def _():
                pltpu.sync_copy(zeros_hbm.at[pl.ds(0, ZROWS)],
                                acc.at[pl.ds(s * ZROWS, ZROWS)])

            @pl.when(s == NSUB - 1)
            def _():
                pltpu.sync_copy(zeros_hbm,
                                acc.at[pl.ds((NSUB - 1) * ZROWS, ZTAIL)])

            plsc.subcore_barrier()

        def round_body(r, carry):
            j0 = r * NBUF
            for b in range(NBUF):
                _wait(sgs[b], bufs[b])
                pltpu.async_copy(bufs[b], acc.at[dst_v.at[j0 + b]], sss[b],
                                 add=True)
            for b in range(NBUF):
                _wait(sss[b], bufs[b])

                @pl.when(j0 + NBUF + b < IDX_HALF)
                def _():
                    pltpu.async_copy(m_hbm.at[src_v.at[j0 + NBUF + b]],
                                     bufs[b], sgs[b])
            return carry

        lax.fori_loop(0, IDX_HALF // NBUF, round_body, 0)

    plsc.subcore_barrier()

    ob = s * OROWS
    tb = (NSUB - 1) * OROWS

    @pl.when((c == 0) & (s < NSUB - 1))
    def _():
        pltpu.sync_copy(acc.at[pl.ds(ob, OROWS)], out0.at[pl.ds(ob, OROWS)])

    @pl.when((c == 0) & (s == NSUB - 1))
    def _():
        pltpu.sync_copy(acc.at[pl.ds(tb, OTAIL)], out0.at[pl.ds(tb, OTAIL)])

    @pl.when((c == 1) & (s < NSUB - 1))
    def _():
        pltpu.sync_copy(acc.at[pl.ds(ob, OROWS)], out1.at[pl.ds(ob, OROWS)])

    @pl.when((c == 1) & (s == NSUB - 1))
    def _():
        pltpu.sync_copy(acc.at[pl.ds(tb, OTAIL)], out1.at[pl.ds(tb, OTAIL)])


@functools.lru_cache(maxsize=None)
def _get_sc_scatter():
    return pl.kernel(
        _sc_scatter_body,
        out_type=[jax.ShapeDtypeStruct((N, D), jnp.float32),
                  jax.ShapeDtypeStruct((N, D), jnp.float32)],
        mesh=plsc.VectorSubcoreMesh(core_axis_name="c", subcore_axis_name="s",
                                    num_cores=NSC, num_subcores=NSUB),
        scratch_types=[
            pltpu.VMEM((IDX_HALF, CHUNK), jnp.int32),
            pltpu.VMEM((IDX_HALF, CHUNK), jnp.int32),
        ] + [pltpu.VMEM((CHUNK, D), jnp.float32) for _ in range(NBUF)] + [
            pltpu.VMEM_SHARED((NP, D), jnp.float32),
        ] + [pltpu.SemaphoreType.DMA for _ in range(2 * NBUF)],
    )


def _dot_t(x, w):
    return lax.dot_general(x, w, (((1,), (1,)), ((), ())),
                           preferred_element_type=jnp.float32)


def _m_body(h_ref, we_ref, be_ref, o_ref):
    o_ref[...] = _dot_t(h_ref[...], we_ref[...]) + be_ref[...]


def _gru_core(a, h, wi_ref, wh_ref, bi_ref, bh_ref):
    gi = _dot_t(a, wi_ref[...]) + bi_ref[...]
    gh = _dot_t(h, wh_ref[...]) + bh_ref[...]
    r = jax.nn.sigmoid(gi[:, :D] + gh[:, :D])
    z = jax.nn.sigmoid(gi[:, D:2 * D] + gh[:, D:2 * D])
    n = jnp.tanh(gi[:, 2 * D:] + r * gh[:, 2 * D:])
    return (1.0 - z) * n + z * h


def _gru_m_body(a0_ref, a1_ref, h_ref, wi_ref, wh_ref, bi_ref, bh_ref,
                wen_ref, ben_ref, h_out, m_out):
    hn = _gru_core(a0_ref[...] + a1_ref[...], h_ref[...],
                   wi_ref, wh_ref, bi_ref, bh_ref)
    h_out[...] = hn
    m_out[...] = _dot_t(hn, wen_ref[...]) + ben_ref[...]


def _gru_fc_body(a0_ref, a1_ref, h_ref, wi_ref, wh_ref, bi_ref, bh_ref,
                 wfc_ref, bfc_ref, o_ref):
    hn = _gru_core(a0_ref[...] + a1_ref[...], h_ref[...],
                   wi_ref, wh_ref, bi_ref, bh_ref)
    o_ref[...] = _dot_t(hn, wfc_ref[...]) + bfc_ref[...]


def _row_spec(width):
    return pl.BlockSpec((R, width), lambda i: (i, 0))


def _full_spec(shape):
    nd = len(shape)
    return pl.BlockSpec(shape, lambda i: (0,) * nd)


def _tc_m(h, we, be):
    return pl.pallas_call(
        _m_body,
        grid=(GRID,),
        in_specs=[_row_spec(D), _full_spec((D, D)), _full_spec((1, D))],
        out_specs=_row_spec(D),
        out_shape=jax.ShapeDtypeStruct((N, D), jnp.float32),
    )(h, we, be)


def _tc_gru_m(a0, a1, h, wi, wh, bi, bh, wen, ben):
    return pl.pallas_call(
        _gru_m_body,
        grid=(GRID,),
        in_specs=[_row_spec(D), _row_spec(D), _row_spec(D),
                  _full_spec((3 * D, D)), _full_spec((3 * D, D)),
                  _full_spec((1, 3 * D)), _full_spec((1, 3 * D)),
                  _full_spec((D, D)), _full_spec((1, D))],
        out_specs=[_row_spec(D), _row_spec(D)],
        out_shape=[jax.ShapeDtypeStruct((N, D), jnp.float32),
                   jax.ShapeDtypeStruct((N, D), jnp.float32)],
    )(a0, a1, h, wi, wh, bi, bh, wen, ben)


def _tc_gru_fc(a0, a1, h, wi, wh, bi, bh, wfc, bfc):
    return pl.pallas_call(
        _gru_fc_body,
        grid=(GRID,),
        in_specs=[_row_spec(D), _row_spec(D), _row_spec(D),
                  _full_spec((3 * D, D)), _full_spec((3 * D, D)),
                  _full_spec((1, 3 * D)), _full_spec((1, 3 * D)),
                  _full_spec((16, D)), _full_spec((1, 16))],
        out_specs=_row_spec(16),
        out_shape=jax.ShapeDtypeStruct((N, 16), jnp.float32),
    )(a0, a1, h, wi, wh, bi, bh, wfc, bfc)


def kernel(features, edge_index, W_e0, b_e0, Wi0, Wh0, bi0, bh0,
           W_e1, b_e1, Wi1, Wh1, bi1, bh1, W_fc, b_fc):
    pad = E_PAD - E
    pad_ids = jnp.arange(pad, dtype=jnp.int32)
    src = jnp.concatenate([edge_index[0], pad_ids % N]).reshape(IDX_ROWS, CHUNK)
    dst = jnp.concatenate([edge_index[1], N + (pad_ids % NSUB)]
                          ).reshape(IDX_ROWS, CHUNK)
    zeros_init = jnp.zeros((ZTAIL, D), jnp.float32)

    params = ((W_e0, b_e0.reshape(1, D), Wi0, Wh0,
               bi0.reshape(1, 3 * D), bh0.reshape(1, 3 * D)),
              (W_e1, b_e1.reshape(1, D), Wi1, Wh1,
               bi1.reshape(1, 3 * D), bh1.reshape(1, 3 * D)))

    h = features
    m = _tc_m(h, params[0][0], params[0][1])
    for layer in range(2):
        _, _, wi, wh, bi, bh = params[layer]
        for step in range(3):
            a0, a1 = _get_sc_scatter()(m, src, dst, zeros_init)
            last = layer == 1 and step == 2
            if last:
                return _tc_gru_fc(a0, a1, h, wi, wh, bi, bh,
                                  W_fc, b_fc.reshape(1, 16))
            if step == 2:
                wen, ben = params[layer + 1][0], params[layer + 1][1]
            else:
                wen, ben = params[layer][0], params[layer][1]
            h, m = _tc_gru_m(a0, a1, h, wi, wh, bi, bh, wen, ben)
